# Initial kernel scaffold; baseline (speedup 1.0000x reference)
import jax
import jax.numpy as jnp
from jax.experimental import pallas as pl


def kernel(x, edge_index, edge_attr, batch, edge_batch, eps, params):
    raise NotImplementedError("write your pallas kernel here")


# speedup vs baseline: 2.6179x; 2.6179x over previous
import functools

import jax
import jax.numpy as jnp
from jax import lax
from jax.experimental import pallas as pl
from jax.experimental.pallas import tpu as pltpu
from jax.experimental.pallas import tpu_sc as plsc

_NC = 2
_NS = 16
_NW = _NC * _NS
_CHUNK = 128

def _mesh():
    return plsc.VectorSubcoreMesh(core_axis_name="c", subcore_axis_name="s",
                                  num_cores=_NC, num_subcores=_NS)


@functools.lru_cache(maxsize=None)
def _gather_rows(E, D, N):
    per_w = E // _NW
    n_ch = per_w // _CHUNK

    @functools.partial(
        pl.kernel,
        out_type=jax.ShapeDtypeStruct((E, D), jnp.float32),
        mesh=_mesh(),
        scratch_types=[
            pltpu.VMEM((n_ch, _CHUNK), jnp.int32),
            pltpu.VMEM((per_w, D), jnp.float32),
            pltpu.SemaphoreType.DMA,
        ],
        compiler_params=pltpu.CompilerParams(use_tc_tiling_on_sc=False),
    )
    def gk(table_hbm, idx_hbm, out_hbm, idx_v, rows_v, sem):
        c = lax.axis_index("c")
        s = lax.axis_index("s")
        w = c * _NS + s
        base = w * per_w
        pltpu.sync_copy(idx_hbm.at[pl.ds(w * n_ch, n_ch)], idx_v)
        cps = [
            pltpu.async_copy(table_hbm.at[idx_v.at[j]],
                             rows_v.at[pl.ds(j * _CHUNK, _CHUNK)], sem)
            for j in range(n_ch)
        ]
        for cp in cps:
            cp.wait()
        pltpu.sync_copy(rows_v, out_hbm.at[pl.ds(base, per_w)])

    return gk


@functools.lru_cache(maxsize=None)
def _scatter_add_rows(E, D, N):
    per_w = E // _NW
    n_ch = per_w // _CHUNK
    load = min(per_w, 512)
    n_load = per_w // load
    n_in = load // _CHUNK
    stripe = N // _NS
    zr = min(stripe, 256)
    wb = min(stripe, 256)
    buf_rows = max(load, zr, wb)

    @functools.partial(
        pl.kernel,
        out_type=jax.ShapeDtypeStruct((_NC * N, D), jnp.float32),
        mesh=_mesh(),
        scratch_types=[
            pltpu.VMEM((n_ch, _CHUNK), jnp.int32),
            pltpu.VMEM((buf_rows, D), jnp.float32),
            pltpu.VMEM_SHARED((N, D), jnp.float32),
        ],
        compiler_params=pltpu.CompilerParams(use_tc_tiling_on_sc=False),
    )
    def sk(data_hbm, idx_hbm, zblk_hbm, out_hbm, idx_v, rows_v, acc):
        c = lax.axis_index("c")
        s = lax.axis_index("s")
        w = c * _NS + s
        base = w * per_w
        pltpu.sync_copy(zblk_hbm, rows_v.at[pl.ds(0, zr)])
        for t in range(stripe // zr):
            pltpu.sync_copy(rows_v.at[pl.ds(0, zr)],
                            acc.at[pl.ds(s * stripe + t * zr, zr)])
        plsc.subcore_barrier()
        pltpu.sync_copy(idx_hbm.at[pl.ds(w * n_ch, n_ch)], idx_v)
        for t in range(n_load):
            pltpu.sync_copy(data_hbm.at[pl.ds(base + t * load, load)],
                            rows_v.at[pl.ds(0, load)])
            for j in range(n_in):
                pltpu.sync_copy(rows_v.at[pl.ds(j * _CHUNK, _CHUNK)],
                                acc.at[idx_v.at[t * n_in + j]], add=True)
        plsc.subcore_barrier()
        for t in range(stripe // wb):
            r0 = s * stripe + t * wb
            pltpu.sync_copy(acc.at[pl.ds(r0, wb)], rows_v.at[pl.ds(0, wb)])
            pltpu.sync_copy(rows_v.at[pl.ds(0, wb)],
                            out_hbm.at[pl.ds(c * N + r0, wb)])

    return sk


@functools.lru_cache(maxsize=None)
def _edge_combine(E, in_c, out_c, Eb=512):
    grid = E // Eb

    def body(xg_ref, ea_ref, w_ref, o_ref):
        wmbf = jnp.dot(w_ref[...], ea_ref[...],
                       preferred_element_type=jnp.float32).astype(jnp.bfloat16)
        xgt = xg_ref[...].T.astype(jnp.bfloat16).astype(jnp.float32)
        acc = xgt[0:1, :] * wmbf[0:out_c, :].astype(jnp.float32)
        for i in range(1, in_c):
            acc = acc + xgt[i:i + 1, :] * \
                wmbf[i * out_c:(i + 1) * out_c, :].astype(jnp.float32)
        o_ref[...] = acc.T

    return pl.pallas_call(
        body,
        grid=(grid,),
        in_specs=[
            pl.BlockSpec((Eb, in_c), lambda i: (i, 0)),
            pl.BlockSpec((5, Eb), lambda i: (0, i)),
            pl.BlockSpec((in_c * out_c, 5), lambda i: (0, 0)),
        ],
        out_specs=pl.BlockSpec((Eb, out_c), lambda i: (i, 0)),
        out_shape=jax.ShapeDtypeStruct((E, out_c), jnp.float32),
    )


@functools.lru_cache(maxsize=None)
def _bn_relu(N, in_c, D, Nb=2048):
    nb = N // Nb

    def body(parts_ref, x_ref, wr_ref, br_ref, g_ref, b_ref, o_ref, stats_ref):
        ph = pl.program_id(0)
        j = pl.program_id(1)
        agg = (parts_ref[0] + parts_ref[1]
               + jnp.dot(x_ref[...], wr_ref[...], preferred_element_type=jnp.float32)
               + br_ref[...])

        @pl.when((ph == 0) & (j == 0))
        def _init():
            stats_ref[...] = jnp.zeros_like(stats_ref)

        @pl.when(ph == 0)
        def _acc():
            stats_ref[0:1, :] += jnp.sum(agg, axis=0, keepdims=True)
            stats_ref[1:2, :] += jnp.sum(agg * agg, axis=0, keepdims=True)

        @pl.when(ph == 1)
        def _apply():
            mu = stats_ref[0:1, :] * (1.0 / N)
            var = stats_ref[1:2, :] * (1.0 / N) - mu * mu
            h = (agg - mu) * lax.rsqrt(var + 1e-5) * g_ref[...] + b_ref[...]
            o_ref[...] = jnp.maximum(h, 0.0)

    return pl.pallas_call(
        body,
        grid=(2, nb),
        in_specs=[
            pl.BlockSpec((2, Nb, D), lambda p, j: (0, j, 0)),
            pl.BlockSpec((Nb, in_c), lambda p, j: (j, 0)),
            pl.BlockSpec((in_c, D), lambda p, j: (0, 0)),
            pl.BlockSpec((1, D), lambda p, j: (0, 0)),
            pl.BlockSpec((1, D), lambda p, j: (0, 0)),
            pl.BlockSpec((1, D), lambda p, j: (0, 0)),
        ],
        out_specs=pl.BlockSpec((Nb, D), lambda p, j: (j, 0)),
        out_shape=jax.ShapeDtypeStruct((N, D), jnp.float32),
        scratch_shapes=[pltpu.VMEM((8, D), jnp.float32)],
    )


@functools.lru_cache(maxsize=None)
def _bn_fx(N, in_c, D, OUT, Nb=2048):
    nb = N // Nb

    def body(parts_ref, x_ref, wr_ref, br_ref, g_ref, b_ref,
             ws_ref, bs_ref, wt_ref, bt_ref, o_ref, stats_ref):
        ph = pl.program_id(0)
        j = pl.program_id(1)
        agg = (parts_ref[0] + parts_ref[1]
               + jnp.dot(x_ref[...], wr_ref[...], preferred_element_type=jnp.float32)
               + br_ref[...])

        @pl.when((ph == 0) & (j == 0))
        def _init():
            stats_ref[...] = jnp.zeros_like(stats_ref)

        @pl.when(ph == 0)
        def _acc():
            stats_ref[0:1, :] += jnp.sum(agg, axis=0, keepdims=True)
            stats_ref[1:2, :] += jnp.sum(agg * agg, axis=0, keepdims=True)

        @pl.when(ph == 1)
        def _apply():
            mu = stats_ref[0:1, :] * (1.0 / N)
            var = stats_ref[1:2, :] * (1.0 / N) - mu * mu
            h2 = (agg - mu) * lax.rsqrt(var + 1e-5) * g_ref[...] + b_ref[...]
            h2 = jnp.maximum(h2, 0.0)
            s = jnp.dot(h2, ws_ref[...], preferred_element_type=jnp.float32) + bs_ref[...]
            s = jnp.clip(s, -30.0, 30.0)
            t = jnp.dot(h2, wt_ref[...], preferred_element_type=jnp.float32) + bt_ref[...]
            o_ref[...] = (1.0 / (1.0 + jnp.exp(s))) * jnp.tanh(t)

    return pl.pallas_call(
        body,
        grid=(2, nb),
        in_specs=[
            pl.BlockSpec((2, Nb, D), lambda p, j: (0, j, 0)),
            pl.BlockSpec((Nb, in_c), lambda p, j: (j, 0)),
            pl.BlockSpec((in_c, D), lambda p, j: (0, 0)),
            pl.BlockSpec((1, D), lambda p, j: (0, 0)),
            pl.BlockSpec((1, D), lambda p, j: (0, 0)),
            pl.BlockSpec((1, D), lambda p, j: (0, 0)),
            pl.BlockSpec((D, OUT), lambda p, j: (0, 0)),
            pl.BlockSpec((1, OUT), lambda p, j: (0, 0)),
            pl.BlockSpec((D, OUT), lambda p, j: (0, 0)),
            pl.BlockSpec((1, OUT), lambda p, j: (0, 0)),
        ],
        out_specs=pl.BlockSpec((Nb, OUT), lambda p, j: (j, 0)),
        out_shape=jax.ShapeDtypeStruct((N, OUT), jnp.float32),
        scratch_shapes=[pltpu.VMEM((8, D), jnp.float32)],
    )


@functools.lru_cache(maxsize=None)
def _decoder_trunk(B, FEAT, ZD, H0, H1d, H2d):
    def body(featp_ref, eps_ref,
             w11_ref, b11_ref, w12_ref, b12_ref,
             wd0_ref, bd0_ref, gd0_ref, td0_ref,
             wd1_ref, bd1_ref, gd1_ref, td1_ref,
             wd2_ref, bd2_ref, gd2_ref, td2_ref,
             z_ref, zmu_ref, zls_ref):
        feat = featp_ref[0] + featp_ref[1]
        zmu = jnp.dot(feat, w11_ref[...], preferred_element_type=jnp.float32) + b11_ref[...]
        zls = jnp.dot(feat, w12_ref[...], preferred_element_type=jnp.float32) + b12_ref[...]
        z = eps_ref[...] * jnp.exp(0.5 * zls) + zmu
        for w_r, b_r, g_r, t_r in ((wd0_ref, bd0_ref, gd0_ref, td0_ref),
                                   (wd1_ref, bd1_ref, gd1_ref, td1_ref),
                                   (wd2_ref, bd2_ref, gd2_ref, td2_ref)):
            u = jnp.dot(z, w_r[...], preferred_element_type=jnp.float32) + b_r[...]
            mu = jnp.mean(u, axis=0, keepdims=True)
            var = jnp.mean(u * u, axis=0, keepdims=True) - mu * mu
            z = jnp.maximum((u - mu) * lax.rsqrt(var + 1e-5) * g_r[...] + t_r[...], 0.0)
        z_ref[...] = z
        zmu_ref[...] = zmu
        zls_ref[...] = zls

    return pl.pallas_call(
        body,
        out_shape=(
            jax.ShapeDtypeStruct((B, H2d), jnp.float32),
            jax.ShapeDtypeStruct((B, ZD), jnp.float32),
            jax.ShapeDtypeStruct((B, ZD), jnp.float32),
        ),
    )


@functools.lru_cache(maxsize=None)
def _decoder_heads(B, HD, DX, DA, DE, Bb=256):
    grid = B // Bb

    def body(z_ref, wx_ref, bx_ref, wa_ref, ba_ref, we_ref, be_ref,
             xr_ref, a_ref, er_ref):
        z = z_ref[...]
        xr_ref[...] = jnp.dot(z, wx_ref[...], preferred_element_type=jnp.float32) + bx_ref[...]
        a_ref[...] = jnp.dot(z, wa_ref[...], preferred_element_type=jnp.float32) + ba_ref[...]
        er_ref[...] = jnp.dot(z, we_ref[...], preferred_element_type=jnp.float32) + be_ref[...]

    return pl.pallas_call(
        body,
        grid=(grid,),
        in_specs=[
            pl.BlockSpec((Bb, HD), lambda i: (i, 0)),
            pl.BlockSpec((HD, DX), lambda i: (0, 0)),
            pl.BlockSpec((1, DX), lambda i: (0, 0)),
            pl.BlockSpec((HD, DA), lambda i: (0, 0)),
            pl.BlockSpec((1, DA), lambda i: (0, 0)),
            pl.BlockSpec((HD, DE), lambda i: (0, 0)),
            pl.BlockSpec((1, DE), lambda i: (0, 0)),
        ],
        out_specs=(
            pl.BlockSpec((Bb, DX), lambda i: (i, 0)),
            pl.BlockSpec((Bb, DA), lambda i: (i, 0)),
            pl.BlockSpec((Bb, DE), lambda i: (i, 0)),
        ),
        out_shape=(
            jax.ShapeDtypeStruct((B, DX), jnp.float32),
            jax.ShapeDtypeStruct((B, DA), jnp.float32),
            jax.ShapeDtypeStruct((B, DE), jnp.float32),
        ),
    )


def kernel(x, edge_index, edge_attr, batch, edge_batch, eps, params):
    p = params
    N, IN = x.shape
    E = edge_attr.shape[0]
    B, ZD = eps.shape
    H1 = p["Wr1"].shape[1]
    H2 = p["Wr2"].shape[1]
    OUT = p["Ws"].shape[1]
    H2d = p["WX"].shape[0]
    DX = p["WX"].shape[1]
    DA = p["WA"].shape[1]
    DE = p["WE"].shape[1]

    src = edge_index[0].reshape(-1, _CHUNK)
    dst = edge_index[1].reshape(-1, _CHUNK)
    bat = batch.reshape(-1, _CHUNK)

    ea5 = jnp.concatenate([edge_attr.T, jnp.ones((1, E), jnp.float32)], axis=0)
    we1t = jnp.concatenate([p["We1"].T, p["be1"].reshape(-1, 1)], axis=1)
    we2t = jnp.concatenate([p["We2"].T, p["be2"].reshape(-1, 1)], axis=1)

    z1 = jnp.zeros((min(N // _NS, 256), H1), jnp.float32)
    z2 = jnp.zeros((min(N // _NS, 256), H2), jnp.float32)
    z3 = jnp.zeros((min(B // _NS, 256), OUT), jnp.float32)

    r2 = lambda a: a.reshape(1, -1)

    xg = _gather_rows(E, IN, N)(x, src)
    m1 = _edge_combine(E, IN, H1)(xg, ea5, we1t)
    part1 = _scatter_add_rows(E, H1, N)(m1, dst, z1).reshape(2, N, H1)
    h1 = _bn_relu(N, IN, H1)(part1, x, p["Wr1"], r2(p["br1"]), r2(p["g1"]), r2(p["b1"]))

    hg = _gather_rows(E, H1, N)(h1, src)
    m2 = _edge_combine(E, H1, H2)(hg, ea5, we2t)
    part2 = _scatter_add_rows(E, H2, N)(m2, dst, z2).reshape(2, N, H2)
    f_x = _bn_fx(N, H1, H2, OUT)(part2, h1, p["Wr2"], r2(p["br2"]), r2(p["g2"]),
                                 r2(p["b2"]), p["Ws"], r2(p["bs"]), p["Wt"], r2(p["bt"]))

    featp = _scatter_add_rows(N, OUT, B)(f_x, bat, z3).reshape(2, B, OUT)
    z, z_mu, z_lsgms = _decoder_trunk(B, OUT, ZD, *[p["Wd%d" % i].shape[1] for i in range(3)])(
        featp, eps,
        p["W11"], r2(p["b11"]), p["W12"], r2(p["b12"]),
        p["Wd0"], r2(p["bd0"]), r2(p["gd0"]), r2(p["betad0"]),
        p["Wd1"], r2(p["bd1"]), r2(p["gd1"]), r2(p["betad1"]),
        p["Wd2"], r2(p["bd2"]), r2(p["gd2"]), r2(p["betad2"]))
    xr, A, Er = _decoder_heads(B, H2d, DX, DA, DE)(
        z, p["WX"], r2(p["bX"]), p["WA"], r2(p["bA"]), p["WE"], r2(p["bE"]))

    return (xr.reshape(B, 38, 16), A, Er.reshape(B, DA, 4), z_mu, z_lsgms)


# speedup vs baseline: 2.9866x; 1.1408x over previous
import functools

import jax
import jax.numpy as jnp
from jax import lax
from jax.experimental import pallas as pl
from jax.experimental.pallas import tpu as pltpu
from jax.experimental.pallas import tpu_sc as plsc

_NC = 2
_NS = 16
_NW = _NC * _NS
_CHUNK = 128

def _mesh():
    return plsc.VectorSubcoreMesh(core_axis_name="c", subcore_axis_name="s",
                                  num_cores=_NC, num_subcores=_NS)


@functools.lru_cache(maxsize=None)
def _gather_rows(E, D, N):
    per_w = E // _NW
    n_ch = per_w // _CHUNK

    @functools.partial(
        pl.kernel,
        out_type=jax.ShapeDtypeStruct((E, D), jnp.float32),
        mesh=_mesh(),
        scratch_types=[
            pltpu.VMEM((n_ch, _CHUNK), jnp.int32),
            pltpu.VMEM((per_w, D), jnp.float32),
            pltpu.SemaphoreType.DMA,
        ],
        compiler_params=pltpu.CompilerParams(use_tc_tiling_on_sc=False),
    )
    def gk(table_hbm, idx_hbm, out_hbm, idx_v, rows_v, sem):
        c = lax.axis_index("c")
        s = lax.axis_index("s")
        w = c * _NS + s
        base = w * per_w
        pltpu.sync_copy(idx_hbm.at[pl.ds(w * n_ch, n_ch)], idx_v)
        cps = [
            pltpu.async_copy(table_hbm.at[idx_v.at[j]],
                             rows_v.at[pl.ds(j * _CHUNK, _CHUNK)], sem)
            for j in range(n_ch)
        ]
        for cp in cps:
            cp.wait()
        pltpu.sync_copy(rows_v, out_hbm.at[pl.ds(base, per_w)])

    return gk


@functools.lru_cache(maxsize=None)
def _scatter_add_rows(E, D, N):
    per_w = E // _NW
    n_ch = per_w // _CHUNK
    load = min(per_w, 512)
    n_load = per_w // load
    n_in = load // _CHUNK
    stripe = N // _NS
    zr = min(stripe, 256)
    wb = min(stripe, 256)
    buf_rows = max(load, zr, wb)

    @functools.partial(
        pl.kernel,
        out_type=jax.ShapeDtypeStruct((_NC * N, D), jnp.float32),
        mesh=_mesh(),
        scratch_types=[
            pltpu.VMEM((n_ch, _CHUNK), jnp.int32),
            pltpu.VMEM((buf_rows, D), jnp.float32),
            pltpu.VMEM_SHARED((N, D), jnp.float32),
        ],
        compiler_params=pltpu.CompilerParams(use_tc_tiling_on_sc=False),
    )
    def sk(data_hbm, idx_hbm, zblk_hbm, out_hbm, idx_v, rows_v, acc):
        c = lax.axis_index("c")
        s = lax.axis_index("s")
        w = c * _NS + s
        base = w * per_w
        pltpu.sync_copy(zblk_hbm, rows_v.at[pl.ds(0, zr)])
        for t in range(stripe // zr):
            pltpu.sync_copy(rows_v.at[pl.ds(0, zr)],
                            acc.at[pl.ds(s * stripe + t * zr, zr)])
        plsc.subcore_barrier()
        pltpu.sync_copy(idx_hbm.at[pl.ds(w * n_ch, n_ch)], idx_v)
        for t in range(n_load):
            pltpu.sync_copy(data_hbm.at[pl.ds(base + t * load, load)],
                            rows_v.at[pl.ds(0, load)])
            for j in range(n_in):
                pltpu.sync_copy(rows_v.at[pl.ds(j * _CHUNK, _CHUNK)],
                                acc.at[idx_v.at[t * n_in + j]], add=True)
        plsc.subcore_barrier()
        for t in range(stripe // wb):
            r0 = s * stripe + t * wb
            pltpu.sync_copy(acc.at[pl.ds(r0, wb)], rows_v.at[pl.ds(0, wb)])
            pltpu.sync_copy(rows_v.at[pl.ds(0, wb)],
                            out_hbm.at[pl.ds(c * N + r0, wb)])

    return sk


@functools.lru_cache(maxsize=None)
def _edge_combine(E, in_c, out_c, Eb=512):
    grid = E // Eb

    def body(xg_ref, ea_ref, w_ref, o_ref):
        wmbf = jnp.dot(w_ref[...], ea_ref[...],
                       preferred_element_type=jnp.float32).astype(jnp.bfloat16)
        xgt = xg_ref[...].T.astype(jnp.bfloat16).astype(jnp.float32)
        acc = xgt[0:1, :] * wmbf[0:out_c, :].astype(jnp.float32)
        for i in range(1, in_c):
            acc = acc + xgt[i:i + 1, :] * \
                wmbf[i * out_c:(i + 1) * out_c, :].astype(jnp.float32)
        o_ref[...] = acc.T

    return pl.pallas_call(
        body,
        grid=(grid,),
        in_specs=[
            pl.BlockSpec((Eb, in_c), lambda i: (i, 0)),
            pl.BlockSpec((5, Eb), lambda i: (0, i)),
            pl.BlockSpec((in_c * out_c, 5), lambda i: (0, 0)),
        ],
        out_specs=pl.BlockSpec((Eb, out_c), lambda i: (i, 0)),
        out_shape=jax.ShapeDtypeStruct((E, out_c), jnp.float32),
    )


@functools.lru_cache(maxsize=None)
def _bn_relu(N, in_c, D, Nb=2048):
    nb = N // Nb

    def body(parts_ref, x_ref, wr_ref, br_ref, g_ref, b_ref, o_ref, stats_ref):
        ph = pl.program_id(0)
        j = pl.program_id(1)
        agg = (parts_ref[0] + parts_ref[1]
               + jnp.dot(x_ref[...], wr_ref[...], preferred_element_type=jnp.float32)
               + br_ref[...])

        @pl.when((ph == 0) & (j == 0))
        def _init():
            stats_ref[...] = jnp.zeros_like(stats_ref)

        @pl.when(ph == 0)
        def _acc():
            stats_ref[0:1, :] += jnp.sum(agg, axis=0, keepdims=True)
            stats_ref[1:2, :] += jnp.sum(agg * agg, axis=0, keepdims=True)

        @pl.when(ph == 1)
        def _apply():
            mu = stats_ref[0:1, :] * (1.0 / N)
            var = stats_ref[1:2, :] * (1.0 / N) - mu * mu
            h = (agg - mu) * lax.rsqrt(var + 1e-5) * g_ref[...] + b_ref[...]
            o_ref[...] = jnp.maximum(h, 0.0)

    return pl.pallas_call(
        body,
        grid=(2, nb),
        in_specs=[
            pl.BlockSpec((2, Nb, D), lambda p, j: (0, j, 0)),
            pl.BlockSpec((Nb, in_c), lambda p, j: (j, 0)),
            pl.BlockSpec((in_c, D), lambda p, j: (0, 0)),
            pl.BlockSpec((1, D), lambda p, j: (0, 0)),
            pl.BlockSpec((1, D), lambda p, j: (0, 0)),
            pl.BlockSpec((1, D), lambda p, j: (0, 0)),
        ],
        out_specs=pl.BlockSpec((Nb, D), lambda p, j: (j, 0)),
        out_shape=jax.ShapeDtypeStruct((N, D), jnp.float32),
        scratch_shapes=[pltpu.VMEM((8, D), jnp.float32)],
    )


@functools.lru_cache(maxsize=None)
def _bn_fx(N, in_c, D, OUT, Nb=2048):
    nb = N // Nb

    def body(parts_ref, x_ref, wr_ref, br_ref, g_ref, b_ref,
             ws_ref, bs_ref, wt_ref, bt_ref, o_ref, stats_ref):
        ph = pl.program_id(0)
        j = pl.program_id(1)
        agg = (parts_ref[0] + parts_ref[1]
               + jnp.dot(x_ref[...], wr_ref[...], preferred_element_type=jnp.float32)
               + br_ref[...])

        @pl.when((ph == 0) & (j == 0))
        def _init():
            stats_ref[...] = jnp.zeros_like(stats_ref)

        @pl.when(ph == 0)
        def _acc():
            stats_ref[0:1, :] += jnp.sum(agg, axis=0, keepdims=True)
            stats_ref[1:2, :] += jnp.sum(agg * agg, axis=0, keepdims=True)

        @pl.when(ph == 1)
        def _apply():
            mu = stats_ref[0:1, :] * (1.0 / N)
            var = stats_ref[1:2, :] * (1.0 / N) - mu * mu
            h2 = (agg - mu) * lax.rsqrt(var + 1e-5) * g_ref[...] + b_ref[...]
            h2 = jnp.maximum(h2, 0.0)
            s = jnp.dot(h2, ws_ref[...], preferred_element_type=jnp.float32) + bs_ref[...]
            s = jnp.clip(s, -30.0, 30.0)
            t = jnp.dot(h2, wt_ref[...], preferred_element_type=jnp.float32) + bt_ref[...]
            o_ref[...] = (1.0 / (1.0 + jnp.exp(s))) * jnp.tanh(t)

    return pl.pallas_call(
        body,
        grid=(2, nb),
        in_specs=[
            pl.BlockSpec((2, Nb, D), lambda p, j: (0, j, 0)),
            pl.BlockSpec((Nb, in_c), lambda p, j: (j, 0)),
            pl.BlockSpec((in_c, D), lambda p, j: (0, 0)),
            pl.BlockSpec((1, D), lambda p, j: (0, 0)),
            pl.BlockSpec((1, D), lambda p, j: (0, 0)),
            pl.BlockSpec((1, D), lambda p, j: (0, 0)),
            pl.BlockSpec((D, OUT), lambda p, j: (0, 0)),
            pl.BlockSpec((1, OUT), lambda p, j: (0, 0)),
            pl.BlockSpec((D, OUT), lambda p, j: (0, 0)),
            pl.BlockSpec((1, OUT), lambda p, j: (0, 0)),
        ],
        out_specs=pl.BlockSpec((Nb, OUT), lambda p, j: (j, 0)),
        out_shape=jax.ShapeDtypeStruct((N, OUT), jnp.float32),
        scratch_shapes=[pltpu.VMEM((8, D), jnp.float32)],
    )


@functools.lru_cache(maxsize=None)
def _decoder_trunk(B, FEAT, ZD, H0, H1d, H2d):
    def body(featp_ref, eps_ref,
             w11_ref, b11_ref, w12_ref, b12_ref,
             wd0_ref, bd0_ref, gd0_ref, td0_ref,
             wd1_ref, bd1_ref, gd1_ref, td1_ref,
             wd2_ref, bd2_ref, gd2_ref, td2_ref,
             z_ref, zmu_ref, zls_ref):
        feat = featp_ref[0] + featp_ref[1]
        zmu = jnp.dot(feat, w11_ref[...], preferred_element_type=jnp.float32) + b11_ref[...]
        zls = jnp.dot(feat, w12_ref[...], preferred_element_type=jnp.float32) + b12_ref[...]
        z = eps_ref[...] * jnp.exp(0.5 * zls) + zmu
        for w_r, b_r, g_r, t_r in ((wd0_ref, bd0_ref, gd0_ref, td0_ref),
                                   (wd1_ref, bd1_ref, gd1_ref, td1_ref),
                                   (wd2_ref, bd2_ref, gd2_ref, td2_ref)):
            u = jnp.dot(z, w_r[...], preferred_element_type=jnp.float32) + b_r[...]
            mu = jnp.mean(u, axis=0, keepdims=True)
            var = jnp.mean(u * u, axis=0, keepdims=True) - mu * mu
            z = jnp.maximum((u - mu) * lax.rsqrt(var + 1e-5) * g_r[...] + t_r[...], 0.0)
        z_ref[...] = z
        zmu_ref[...] = zmu
        zls_ref[...] = zls

    return pl.pallas_call(
        body,
        out_shape=(
            jax.ShapeDtypeStruct((B, H2d), jnp.float32),
            jax.ShapeDtypeStruct((B, ZD), jnp.float32),
            jax.ShapeDtypeStruct((B, ZD), jnp.float32),
        ),
    )


@functools.lru_cache(maxsize=None)
def _decoder_heads(B, HD, DX, DA, DE, Bb=256):
    grid = B // Bb

    def body(z_ref, wx_ref, bx_ref, wa_ref, ba_ref, we_ref, be_ref,
             xr_ref, a_ref, er_ref):
        z = z_ref[...]
        xr_ref[...] = jnp.dot(z, wx_ref[...], preferred_element_type=jnp.float32) + bx_ref[...]
        a_ref[...] = jnp.dot(z, wa_ref[...], preferred_element_type=jnp.float32) + ba_ref[...]
        er_ref[...] = jnp.dot(z, we_ref[...], preferred_element_type=jnp.float32) + be_ref[...]

    return pl.pallas_call(
        body,
        grid=(grid,),
        in_specs=[
            pl.BlockSpec((Bb, HD), lambda i: (i, 0)),
            pl.BlockSpec((HD, DX), lambda i: (0, 0)),
            pl.BlockSpec((1, DX), lambda i: (0, 0)),
            pl.BlockSpec((HD, DA), lambda i: (0, 0)),
            pl.BlockSpec((1, DA), lambda i: (0, 0)),
            pl.BlockSpec((HD, DE), lambda i: (0, 0)),
            pl.BlockSpec((1, DE), lambda i: (0, 0)),
        ],
        out_specs=(
            pl.BlockSpec((Bb, DX), lambda i: (i, 0)),
            pl.BlockSpec((Bb, DA), lambda i: (i, 0)),
            pl.BlockSpec((Bb, DE), lambda i: (i, 0)),
        ),
        out_shape=(
            jax.ShapeDtypeStruct((B, DX), jnp.float32),
            jax.ShapeDtypeStruct((B, DA), jnp.float32),
            jax.ShapeDtypeStruct((B, DE), jnp.float32),
        ),
    )


def kernel(x, edge_index, edge_attr, batch, edge_batch, eps, params):
    p = params
    N, IN = x.shape
    E = edge_attr.shape[0]
    B, ZD = eps.shape
    H1 = p["Wr1"].shape[1]
    H2 = p["Wr2"].shape[1]
    OUT = p["Ws"].shape[1]
    H2d = p["WX"].shape[0]
    DX = p["WX"].shape[1]
    DA = p["WA"].shape[1]
    DE = p["WE"].shape[1]

    src = edge_index[0].reshape(-1, _CHUNK)
    dst = edge_index[1].reshape(-1, _CHUNK)
    bat = batch.reshape(-1, _CHUNK)

    ea5 = jnp.concatenate([edge_attr.T, jnp.ones((1, E), jnp.float32)], axis=0)
    we1t = jnp.concatenate([p["We1"].T, p["be1"].reshape(-1, 1)], axis=1)
    we2t = jnp.concatenate([p["We2"].T, p["be2"].reshape(-1, 1)], axis=1)

    z1 = jnp.zeros((min(N // _NS, 256), H1), jnp.float32)
    z2 = jnp.zeros((min(N // _NS, 256), H2), jnp.float32)
    z3 = jnp.zeros((min(B // _NS, 256), OUT), jnp.float32)

    r2 = lambda a: a.reshape(1, -1)

    xg = _gather_rows(E, IN, N)(x, src)
    m1 = _edge_combine(E, IN, H1, Eb=1024)(xg, ea5, we1t)
    part1 = _scatter_add_rows(E, H1, N)(m1, dst, z1).reshape(2, N, H1)
    h1 = _bn_relu(N, IN, H1)(part1, x, p["Wr1"], r2(p["br1"]), r2(p["g1"]), r2(p["b1"]))

    hg = _gather_rows(E, H1, N)(h1, src)
    m2 = _edge_combine(E, H1, H2, Eb=1024)(hg, ea5, we2t)
    part2 = _scatter_add_rows(E, H2, N)(m2, dst, z2).reshape(2, N, H2)
    f_x = _bn_fx(N, H1, H2, OUT)(part2, h1, p["Wr2"], r2(p["br2"]), r2(p["g2"]),
                                 r2(p["b2"]), p["Ws"], r2(p["bs"]), p["Wt"], r2(p["bt"]))

    featp = _scatter_add_rows(N, OUT, B)(f_x, bat, z3).reshape(2, B, OUT)
    z, z_mu, z_lsgms = _decoder_trunk(B, OUT, ZD, *[p["Wd%d" % i].shape[1] for i in range(3)])(
        featp, eps,
        p["W11"], r2(p["b11"]), p["W12"], r2(p["b12"]),
        p["Wd0"], r2(p["bd0"]), r2(p["gd0"]), r2(p["betad0"]),
        p["Wd1"], r2(p["bd1"]), r2(p["gd1"]), r2(p["betad1"]),
        p["Wd2"], r2(p["bd2"]), r2(p["gd2"]), r2(p["betad2"]))
    xr, A, Er = _decoder_heads(B, H2d, DX, DA, DE)(
        z, p["WX"], r2(p["bX"]), p["WA"], r2(p["bA"]), p["WE"], r2(p["bE"]))

    return (xr.reshape(B, 38, 16), A, Er.reshape(B, DA, 4), z_mu, z_lsgms)


# speedup vs baseline: 3.1609x; 1.0584x over previous
import functools

import jax
import jax.numpy as jnp
from jax import lax
from jax.experimental import pallas as pl
from jax.experimental.pallas import tpu as pltpu
from jax.experimental.pallas import tpu_sc as plsc

_NC = 2
_NS = 16
_NW = _NC * _NS
_CHUNK = 128

def _mesh():
    return plsc.VectorSubcoreMesh(core_axis_name="c", subcore_axis_name="s",
                                  num_cores=_NC, num_subcores=_NS)


@functools.lru_cache(maxsize=None)
def _gather_rows(E, D, N):
    per_w = E // _NW
    n_ch = per_w // _CHUNK

    @functools.partial(
        pl.kernel,
        out_type=jax.ShapeDtypeStruct((E, D), jnp.float32),
        mesh=_mesh(),
        scratch_types=[
            pltpu.VMEM((n_ch, _CHUNK), jnp.int32),
            pltpu.VMEM((per_w, D), jnp.float32),
            pltpu.SemaphoreType.DMA,
        ],
        compiler_params=pltpu.CompilerParams(use_tc_tiling_on_sc=False),
    )
    def gk(table_hbm, idx_hbm, out_hbm, idx_v, rows_v, sem):
        c = lax.axis_index("c")
        s = lax.axis_index("s")
        w = c * _NS + s
        base = w * per_w
        pltpu.sync_copy(idx_hbm.at[pl.ds(w * n_ch, n_ch)], idx_v)
        cps = [
            pltpu.async_copy(table_hbm.at[idx_v.at[j]],
                             rows_v.at[pl.ds(j * _CHUNK, _CHUNK)], sem)
            for j in range(n_ch)
        ]
        for cp in cps:
            cp.wait()
        pltpu.sync_copy(rows_v, out_hbm.at[pl.ds(base, per_w)])

    return gk


@functools.lru_cache(maxsize=None)
def _scatter_add_rows(E, D, N):
    per_w = E // _NW
    n_ch = per_w // _CHUNK
    load = min(per_w, 512)
    n_load = per_w // load
    n_in = load // _CHUNK
    stripe = N // _NS
    zr = min(stripe, 256)
    wb = min(stripe, 256)
    buf_rows = max(load, zr, wb)

    @functools.partial(
        pl.kernel,
        out_type=jax.ShapeDtypeStruct((_NC * N, D), jnp.float32),
        mesh=_mesh(),
        scratch_types=[
            pltpu.VMEM((n_ch, _CHUNK), jnp.int32),
            pltpu.VMEM((buf_rows, D), jnp.float32),
            pltpu.VMEM_SHARED((N, D), jnp.float32),
        ],
        compiler_params=pltpu.CompilerParams(use_tc_tiling_on_sc=False),
    )
    def sk(data_hbm, idx_hbm, zblk_hbm, out_hbm, idx_v, rows_v, acc):
        c = lax.axis_index("c")
        s = lax.axis_index("s")
        w = c * _NS + s
        base = w * per_w
        pltpu.sync_copy(zblk_hbm, rows_v.at[pl.ds(0, zr)])
        for t in range(stripe // zr):
            pltpu.sync_copy(rows_v.at[pl.ds(0, zr)],
                            acc.at[pl.ds(s * stripe + t * zr, zr)])
        plsc.subcore_barrier()
        pltpu.sync_copy(idx_hbm.at[pl.ds(w * n_ch, n_ch)], idx_v)
        for t in range(n_load):
            pltpu.sync_copy(data_hbm.at[pl.ds(base + t * load, load)],
                            rows_v.at[pl.ds(0, load)])
            for j in range(n_in):
                pltpu.sync_copy(rows_v.at[pl.ds(j * _CHUNK, _CHUNK)],
                                acc.at[idx_v.at[t * n_in + j]], add=True)
        plsc.subcore_barrier()
        for t in range(stripe // wb):
            r0 = s * stripe + t * wb
            pltpu.sync_copy(acc.at[pl.ds(r0, wb)], rows_v.at[pl.ds(0, wb)])
            pltpu.sync_copy(rows_v.at[pl.ds(0, wb)],
                            out_hbm.at[pl.ds(c * N + r0, wb)])

    return sk


@functools.lru_cache(maxsize=None)
def _edge_combine(E, in_c, out_c, Eb=512):
    grid = E // Eb

    def body(xg_ref, ea_ref, w_ref, o_ref):
        wmbf = jnp.dot(w_ref[...], ea_ref[...],
                       preferred_element_type=jnp.float32).astype(jnp.bfloat16)
        xgt = xg_ref[...].T.astype(jnp.bfloat16).astype(jnp.float32)
        acc = xgt[0:1, :] * wmbf[0:out_c, :].astype(jnp.float32)
        for i in range(1, in_c):
            acc = acc + xgt[i:i + 1, :] * \
                wmbf[i * out_c:(i + 1) * out_c, :].astype(jnp.float32)
        o_ref[...] = acc.T

    return pl.pallas_call(
        body,
        grid=(grid,),
        in_specs=[
            pl.BlockSpec((Eb, in_c), lambda i: (i, 0)),
            pl.BlockSpec((5, Eb), lambda i: (0, i)),
            pl.BlockSpec((in_c * out_c, 5), lambda i: (0, 0)),
        ],
        out_specs=pl.BlockSpec((Eb, out_c), lambda i: (i, 0)),
        out_shape=jax.ShapeDtypeStruct((E, out_c), jnp.float32),
    )


@functools.lru_cache(maxsize=None)
def _bn_relu(N, in_c, D, Nb=2048):
    nb = N // Nb

    def body(parts_ref, x_ref, wr_ref, br_ref, g_ref, b_ref, o_ref, stats_ref):
        ph = pl.program_id(0)
        j = pl.program_id(1)
        agg = (parts_ref[0] + parts_ref[1]
               + jnp.dot(x_ref[...], wr_ref[...], preferred_element_type=jnp.float32)
               + br_ref[...])

        @pl.when((ph == 0) & (j == 0))
        def _init():
            stats_ref[...] = jnp.zeros_like(stats_ref)

        @pl.when(ph == 0)
        def _acc():
            stats_ref[0:1, :] += jnp.sum(agg, axis=0, keepdims=True)
            stats_ref[1:2, :] += jnp.sum(agg * agg, axis=0, keepdims=True)

        @pl.when(ph == 1)
        def _apply():
            mu = stats_ref[0:1, :] * (1.0 / N)
            var = stats_ref[1:2, :] * (1.0 / N) - mu * mu
            h = (agg - mu) * lax.rsqrt(var + 1e-5) * g_ref[...] + b_ref[...]
            o_ref[...] = jnp.maximum(h, 0.0)

    return pl.pallas_call(
        body,
        grid=(2, nb),
        in_specs=[
            pl.BlockSpec((2, Nb, D), lambda p, j: (0, j, 0)),
            pl.BlockSpec((Nb, in_c), lambda p, j: (j, 0)),
            pl.BlockSpec((in_c, D), lambda p, j: (0, 0)),
            pl.BlockSpec((1, D), lambda p, j: (0, 0)),
            pl.BlockSpec((1, D), lambda p, j: (0, 0)),
            pl.BlockSpec((1, D), lambda p, j: (0, 0)),
        ],
        out_specs=pl.BlockSpec((Nb, D), lambda p, j: (j, 0)),
        out_shape=jax.ShapeDtypeStruct((N, D), jnp.float32),
        scratch_shapes=[pltpu.VMEM((8, D), jnp.float32)],
    )


@functools.lru_cache(maxsize=None)
def _bn_fx(N, in_c, D, OUT, Nb=2048):
    nb = N // Nb

    def body(parts_ref, x_ref, wr_ref, br_ref, g_ref, b_ref,
             ws_ref, bs_ref, wt_ref, bt_ref, o_ref, stats_ref):
        ph = pl.program_id(0)
        j = pl.program_id(1)
        agg = (parts_ref[0] + parts_ref[1]
               + jnp.dot(x_ref[...], wr_ref[...], preferred_element_type=jnp.float32)
               + br_ref[...])

        @pl.when((ph == 0) & (j == 0))
        def _init():
            stats_ref[...] = jnp.zeros_like(stats_ref)

        @pl.when(ph == 0)
        def _acc():
            stats_ref[0:1, :] += jnp.sum(agg, axis=0, keepdims=True)
            stats_ref[1:2, :] += jnp.sum(agg * agg, axis=0, keepdims=True)

        @pl.when(ph == 1)
        def _apply():
            mu = stats_ref[0:1, :] * (1.0 / N)
            var = stats_ref[1:2, :] * (1.0 / N) - mu * mu
            h2 = (agg - mu) * lax.rsqrt(var + 1e-5) * g_ref[...] + b_ref[...]
            h2 = jnp.maximum(h2, 0.0)
            s = jnp.dot(h2, ws_ref[...], preferred_element_type=jnp.float32) + bs_ref[...]
            s = jnp.clip(s, -30.0, 30.0)
            t = jnp.dot(h2, wt_ref[...], preferred_element_type=jnp.float32) + bt_ref[...]
            o_ref[...] = (1.0 / (1.0 + jnp.exp(s))) * jnp.tanh(t)

    return pl.pallas_call(
        body,
        grid=(2, nb),
        in_specs=[
            pl.BlockSpec((2, Nb, D), lambda p, j: (0, j, 0)),
            pl.BlockSpec((Nb, in_c), lambda p, j: (j, 0)),
            pl.BlockSpec((in_c, D), lambda p, j: (0, 0)),
            pl.BlockSpec((1, D), lambda p, j: (0, 0)),
            pl.BlockSpec((1, D), lambda p, j: (0, 0)),
            pl.BlockSpec((1, D), lambda p, j: (0, 0)),
            pl.BlockSpec((D, OUT), lambda p, j: (0, 0)),
            pl.BlockSpec((1, OUT), lambda p, j: (0, 0)),
            pl.BlockSpec((D, OUT), lambda p, j: (0, 0)),
            pl.BlockSpec((1, OUT), lambda p, j: (0, 0)),
        ],
        out_specs=pl.BlockSpec((Nb, OUT), lambda p, j: (j, 0)),
        out_shape=jax.ShapeDtypeStruct((N, OUT), jnp.float32),
        scratch_shapes=[pltpu.VMEM((8, D), jnp.float32)],
    )


@functools.lru_cache(maxsize=None)
def _decoder_trunk(B, FEAT, ZD, H0, H1d, H2d):
    def body(featp_ref, eps_ref,
             w11_ref, b11_ref, w12_ref, b12_ref,
             wd0_ref, bd0_ref, gd0_ref, td0_ref,
             wd1_ref, bd1_ref, gd1_ref, td1_ref,
             wd2_ref, bd2_ref, gd2_ref, td2_ref,
             z_ref, zmu_ref, zls_ref):
        feat = featp_ref[0] + featp_ref[1]
        zmu = jnp.dot(feat, w11_ref[...], preferred_element_type=jnp.float32) + b11_ref[...]
        zls = jnp.dot(feat, w12_ref[...], preferred_element_type=jnp.float32) + b12_ref[...]
        z = eps_ref[...] * jnp.exp(0.5 * zls) + zmu
        for w_r, b_r, g_r, t_r in ((wd0_ref, bd0_ref, gd0_ref, td0_ref),
                                   (wd1_ref, bd1_ref, gd1_ref, td1_ref),
                                   (wd2_ref, bd2_ref, gd2_ref, td2_ref)):
            u = jnp.dot(z, w_r[...], preferred_element_type=jnp.float32) + b_r[...]
            mu = jnp.mean(u, axis=0, keepdims=True)
            var = jnp.mean(u * u, axis=0, keepdims=True) - mu * mu
            z = jnp.maximum((u - mu) * lax.rsqrt(var + 1e-5) * g_r[...] + t_r[...], 0.0)
        z_ref[...] = z
        zmu_ref[...] = zmu
        zls_ref[...] = zls

    return pl.pallas_call(
        body,
        out_shape=(
            jax.ShapeDtypeStruct((B, H2d), jnp.float32),
            jax.ShapeDtypeStruct((B, ZD), jnp.float32),
            jax.ShapeDtypeStruct((B, ZD), jnp.float32),
        ),
    )


@functools.lru_cache(maxsize=None)
def _decoder_heads(B, HD, DX, DA, DE, Bb=256):
    grid = B // Bb

    def body(z_ref, wx_ref, bx_ref, wa_ref, ba_ref, we_ref, be_ref,
             xr_ref, a_ref, er_ref):
        z = z_ref[...]
        xr_ref[...] = jnp.dot(z, wx_ref[...], preferred_element_type=jnp.float32) + bx_ref[...]
        a_ref[...] = jnp.dot(z, wa_ref[...], preferred_element_type=jnp.float32) + ba_ref[...]
        er_ref[...] = jnp.dot(z, we_ref[...], preferred_element_type=jnp.float32) + be_ref[...]

    return pl.pallas_call(
        body,
        grid=(grid,),
        in_specs=[
            pl.BlockSpec((Bb, HD), lambda i: (i, 0)),
            pl.BlockSpec((HD, DX), lambda i: (0, 0)),
            pl.BlockSpec((1, DX), lambda i: (0, 0)),
            pl.BlockSpec((HD, DA), lambda i: (0, 0)),
            pl.BlockSpec((1, DA), lambda i: (0, 0)),
            pl.BlockSpec((HD, DE), lambda i: (0, 0)),
            pl.BlockSpec((1, DE), lambda i: (0, 0)),
        ],
        out_specs=(
            pl.BlockSpec((Bb, DX), lambda i: (i, 0)),
            pl.BlockSpec((Bb, DA), lambda i: (i, 0)),
            pl.BlockSpec((Bb, DE), lambda i: (i, 0)),
        ),
        out_shape=(
            jax.ShapeDtypeStruct((B, DX), jnp.float32),
            jax.ShapeDtypeStruct((B, DA), jnp.float32),
            jax.ShapeDtypeStruct((B, DE), jnp.float32),
        ),
    )


def kernel(x, edge_index, edge_attr, batch, edge_batch, eps, params):
    p = params
    N, IN = x.shape
    E = edge_attr.shape[0]
    B, ZD = eps.shape
    H1 = p["Wr1"].shape[1]
    H2 = p["Wr2"].shape[1]
    OUT = p["Ws"].shape[1]
    H2d = p["WX"].shape[0]
    DX = p["WX"].shape[1]
    DA = p["WA"].shape[1]
    DE = p["WE"].shape[1]

    src = edge_index[0].reshape(-1, _CHUNK)
    dst = edge_index[1].reshape(-1, _CHUNK)
    bat = batch.reshape(-1, _CHUNK)

    ea5 = jnp.concatenate([edge_attr.T, jnp.ones((1, E), jnp.float32)], axis=0)
    we1t = jnp.concatenate([p["We1"].T, p["be1"].reshape(-1, 1)], axis=1)
    we2t = jnp.concatenate([p["We2"].T, p["be2"].reshape(-1, 1)], axis=1)

    z1 = jnp.zeros((min(N // _NS, 256), H1), jnp.float32)
    z2 = jnp.zeros((min(N // _NS, 256), H2), jnp.float32)
    z3 = jnp.zeros((min(B // _NS, 256), OUT), jnp.float32)

    r2 = lambda a: a.reshape(1, -1)

    xg = _gather_rows(E, IN, N)(x, src)
    m1 = _edge_combine(E, IN, H1, Eb=2048)(xg, ea5, we1t)
    part1 = _scatter_add_rows(E, H1, N)(m1, dst, z1).reshape(2, N, H1)
    h1 = _bn_relu(N, IN, H1)(part1, x, p["Wr1"], r2(p["br1"]), r2(p["g1"]), r2(p["b1"]))

    hg = _gather_rows(E, H1, N)(h1, src)
    m2 = _edge_combine(E, H1, H2, Eb=2048)(hg, ea5, we2t)
    part2 = _scatter_add_rows(E, H2, N)(m2, dst, z2).reshape(2, N, H2)
    f_x = _bn_fx(N, H1, H2, OUT)(part2, h1, p["Wr2"], r2(p["br2"]), r2(p["g2"]),
                                 r2(p["b2"]), p["Ws"], r2(p["bs"]), p["Wt"], r2(p["bt"]))

    featp = _scatter_add_rows(N, OUT, B)(f_x, bat, z3).reshape(2, B, OUT)
    z, z_mu, z_lsgms = _decoder_trunk(B, OUT, ZD, *[p["Wd%d" % i].shape[1] for i in range(3)])(
        featp, eps,
        p["W11"], r2(p["b11"]), p["W12"], r2(p["b12"]),
        p["Wd0"], r2(p["bd0"]), r2(p["gd0"]), r2(p["betad0"]),
        p["Wd1"], r2(p["bd1"]), r2(p["gd1"]), r2(p["betad1"]),
        p["Wd2"], r2(p["bd2"]), r2(p["gd2"]), r2(p["betad2"]))
    xr, A, Er = _decoder_heads(B, H2d, DX, DA, DE)(
        z, p["WX"], r2(p["bX"]), p["WA"], r2(p["bA"]), p["WE"], r2(p["bE"]))

    return (xr.reshape(B, 38, 16), A, Er.reshape(B, DA, 4), z_mu, z_lsgms)


# speedup vs baseline: 3.2530x; 1.0291x over previous
import functools

import jax
import jax.numpy as jnp
from jax import lax
from jax.experimental import pallas as pl
from jax.experimental.pallas import tpu as pltpu
from jax.experimental.pallas import tpu_sc as plsc

_NC = 2
_NS = 16
_NW = _NC * _NS
_CHUNK = 128

def _mesh():
    return plsc.VectorSubcoreMesh(core_axis_name="c", subcore_axis_name="s",
                                  num_cores=_NC, num_subcores=_NS)


@functools.lru_cache(maxsize=None)
def _gather_rows(E, D, N):
    per_w = E // _NW
    n_ch = per_w // _CHUNK

    @functools.partial(
        pl.kernel,
        out_type=jax.ShapeDtypeStruct((E, D), jnp.float32),
        mesh=_mesh(),
        scratch_types=[
            pltpu.VMEM((n_ch, _CHUNK), jnp.int32),
            pltpu.VMEM((per_w, D), jnp.float32),
            pltpu.SemaphoreType.DMA,
        ],
        compiler_params=pltpu.CompilerParams(use_tc_tiling_on_sc=False),
    )
    def gk(table_hbm, idx_hbm, out_hbm, idx_v, rows_v, sem):
        c = lax.axis_index("c")
        s = lax.axis_index("s")
        w = c * _NS + s
        base = w * per_w
        pltpu.sync_copy(idx_hbm.at[pl.ds(w * n_ch, n_ch)], idx_v)
        cps = [
            pltpu.async_copy(table_hbm.at[idx_v.at[j]],
                             rows_v.at[pl.ds(j * _CHUNK, _CHUNK)], sem)
            for j in range(n_ch)
        ]
        for cp in cps:
            cp.wait()
        pltpu.sync_copy(rows_v, out_hbm.at[pl.ds(base, per_w)])

    return gk


@functools.lru_cache(maxsize=None)
def _scatter_add_rows(E, D, N):
    per_w = E // _NW
    n_ch = per_w // _CHUNK
    load = min(per_w, 512)
    n_load = per_w // load
    n_in = load // _CHUNK
    stripe = N // _NS
    zr = min(stripe, 256)
    wb = min(stripe, 256)
    buf_rows = max(load, zr, wb)

    @functools.partial(
        pl.kernel,
        out_type=jax.ShapeDtypeStruct((_NC * N, D), jnp.float32),
        mesh=_mesh(),
        scratch_types=[
            pltpu.VMEM((n_ch, _CHUNK), jnp.int32),
            pltpu.VMEM((buf_rows, D), jnp.float32),
            pltpu.VMEM_SHARED((N, D), jnp.float32),
        ],
        compiler_params=pltpu.CompilerParams(use_tc_tiling_on_sc=False),
    )
    def sk(data_hbm, idx_hbm, zblk_hbm, out_hbm, idx_v, rows_v, acc):
        c = lax.axis_index("c")
        s = lax.axis_index("s")
        w = c * _NS + s
        base = w * per_w
        pltpu.sync_copy(zblk_hbm, rows_v.at[pl.ds(0, zr)])
        for t in range(stripe // zr):
            pltpu.sync_copy(rows_v.at[pl.ds(0, zr)],
                            acc.at[pl.ds(s * stripe + t * zr, zr)])
        plsc.subcore_barrier()
        pltpu.sync_copy(idx_hbm.at[pl.ds(w * n_ch, n_ch)], idx_v)
        for t in range(n_load):
            pltpu.sync_copy(data_hbm.at[pl.ds(base + t * load, load)],
                            rows_v.at[pl.ds(0, load)])
            for j in range(n_in):
                pltpu.sync_copy(rows_v.at[pl.ds(j * _CHUNK, _CHUNK)],
                                acc.at[idx_v.at[t * n_in + j]], add=True)
        plsc.subcore_barrier()
        for t in range(stripe // wb):
            r0 = s * stripe + t * wb
            pltpu.sync_copy(acc.at[pl.ds(r0, wb)], rows_v.at[pl.ds(0, wb)])
            pltpu.sync_copy(rows_v.at[pl.ds(0, wb)],
                            out_hbm.at[pl.ds(c * N + r0, wb)])

    return sk


@functools.lru_cache(maxsize=None)
def _edge_combine(E, in_c, out_c, Eb=512):
    grid = E // Eb

    def body(xg_ref, ea_ref, w_ref, o_ref):
        wmbf = jnp.dot(w_ref[...], ea_ref[...],
                       preferred_element_type=jnp.float32).astype(jnp.bfloat16)
        xgt = xg_ref[...].T.astype(jnp.bfloat16).astype(jnp.float32)
        acc = xgt[0:1, :] * wmbf[0:out_c, :].astype(jnp.float32)
        for i in range(1, in_c):
            acc = acc + xgt[i:i + 1, :] * \
                wmbf[i * out_c:(i + 1) * out_c, :].astype(jnp.float32)
        o_ref[...] = acc.T

    return pl.pallas_call(
        body,
        grid=(grid,),
        in_specs=[
            pl.BlockSpec((Eb, in_c), lambda i: (i, 0)),
            pl.BlockSpec((5, Eb), lambda i: (0, i)),
            pl.BlockSpec((in_c * out_c, 5), lambda i: (0, 0)),
        ],
        out_specs=pl.BlockSpec((Eb, out_c), lambda i: (i, 0)),
        out_shape=jax.ShapeDtypeStruct((E, out_c), jnp.float32),
    )


@functools.lru_cache(maxsize=None)
def _bn_relu(N, in_c, D, Nb=2048):
    nb = N // Nb

    def body(parts_ref, x_ref, wr_ref, br_ref, g_ref, b_ref, o_ref, stats_ref):
        ph = pl.program_id(0)
        j = pl.program_id(1)
        agg = (parts_ref[0] + parts_ref[1]
               + jnp.dot(x_ref[...], wr_ref[...], preferred_element_type=jnp.float32)
               + br_ref[...])

        @pl.when((ph == 0) & (j == 0))
        def _init():
            stats_ref[...] = jnp.zeros_like(stats_ref)

        @pl.when(ph == 0)
        def _acc():
            stats_ref[0:1, :] += jnp.sum(agg, axis=0, keepdims=True)
            stats_ref[1:2, :] += jnp.sum(agg * agg, axis=0, keepdims=True)

        @pl.when(ph == 1)
        def _apply():
            mu = stats_ref[0:1, :] * (1.0 / N)
            var = stats_ref[1:2, :] * (1.0 / N) - mu * mu
            h = (agg - mu) * lax.rsqrt(var + 1e-5) * g_ref[...] + b_ref[...]
            o_ref[...] = jnp.maximum(h, 0.0)

    return pl.pallas_call(
        body,
        grid=(2, nb),
        in_specs=[
            pl.BlockSpec((2, Nb, D), lambda p, j: (0, j, 0)),
            pl.BlockSpec((Nb, in_c), lambda p, j: (j, 0)),
            pl.BlockSpec((in_c, D), lambda p, j: (0, 0)),
            pl.BlockSpec((1, D), lambda p, j: (0, 0)),
            pl.BlockSpec((1, D), lambda p, j: (0, 0)),
            pl.BlockSpec((1, D), lambda p, j: (0, 0)),
        ],
        out_specs=pl.BlockSpec((Nb, D), lambda p, j: (j, 0)),
        out_shape=jax.ShapeDtypeStruct((N, D), jnp.float32),
        scratch_shapes=[pltpu.VMEM((8, D), jnp.float32)],
    )


@functools.lru_cache(maxsize=None)
def _bn_fx(N, in_c, D, OUT, Nb=2048):
    nb = N // Nb

    def body(parts_ref, x_ref, wr_ref, br_ref, g_ref, b_ref,
             ws_ref, bs_ref, wt_ref, bt_ref, o_ref, stats_ref):
        ph = pl.program_id(0)
        j = pl.program_id(1)
        agg = (parts_ref[0] + parts_ref[1]
               + jnp.dot(x_ref[...], wr_ref[...], preferred_element_type=jnp.float32)
               + br_ref[...])

        @pl.when((ph == 0) & (j == 0))
        def _init():
            stats_ref[...] = jnp.zeros_like(stats_ref)

        @pl.when(ph == 0)
        def _acc():
            stats_ref[0:1, :] += jnp.sum(agg, axis=0, keepdims=True)
            stats_ref[1:2, :] += jnp.sum(agg * agg, axis=0, keepdims=True)

        @pl.when(ph == 1)
        def _apply():
            mu = stats_ref[0:1, :] * (1.0 / N)
            var = stats_ref[1:2, :] * (1.0 / N) - mu * mu
            h2 = (agg - mu) * lax.rsqrt(var + 1e-5) * g_ref[...] + b_ref[...]
            h2 = jnp.maximum(h2, 0.0)
            s = jnp.dot(h2, ws_ref[...], preferred_element_type=jnp.float32) + bs_ref[...]
            s = jnp.clip(s, -30.0, 30.0)
            t = jnp.dot(h2, wt_ref[...], preferred_element_type=jnp.float32) + bt_ref[...]
            o_ref[...] = (1.0 / (1.0 + jnp.exp(s))) * jnp.tanh(t)

    return pl.pallas_call(
        body,
        grid=(2, nb),
        in_specs=[
            pl.BlockSpec((2, Nb, D), lambda p, j: (0, j, 0)),
            pl.BlockSpec((Nb, in_c), lambda p, j: (j, 0)),
            pl.BlockSpec((in_c, D), lambda p, j: (0, 0)),
            pl.BlockSpec((1, D), lambda p, j: (0, 0)),
            pl.BlockSpec((1, D), lambda p, j: (0, 0)),
            pl.BlockSpec((1, D), lambda p, j: (0, 0)),
            pl.BlockSpec((D, OUT), lambda p, j: (0, 0)),
            pl.BlockSpec((1, OUT), lambda p, j: (0, 0)),
            pl.BlockSpec((D, OUT), lambda p, j: (0, 0)),
            pl.BlockSpec((1, OUT), lambda p, j: (0, 0)),
        ],
        out_specs=pl.BlockSpec((Nb, OUT), lambda p, j: (j, 0)),
        out_shape=jax.ShapeDtypeStruct((N, OUT), jnp.float32),
        scratch_shapes=[pltpu.VMEM((8, D), jnp.float32)],
    )


@functools.lru_cache(maxsize=None)
def _decoder_trunk(B, FEAT, ZD, H0, H1d, H2d):
    def body(featp_ref, eps_ref,
             w11_ref, b11_ref, w12_ref, b12_ref,
             wd0_ref, bd0_ref, gd0_ref, td0_ref,
             wd1_ref, bd1_ref, gd1_ref, td1_ref,
             wd2_ref, bd2_ref, gd2_ref, td2_ref,
             z_ref, zmu_ref, zls_ref):
        feat = featp_ref[0] + featp_ref[1]
        zmu = jnp.dot(feat, w11_ref[...], preferred_element_type=jnp.float32) + b11_ref[...]
        zls = jnp.dot(feat, w12_ref[...], preferred_element_type=jnp.float32) + b12_ref[...]
        z = eps_ref[...] * jnp.exp(0.5 * zls) + zmu
        for w_r, b_r, g_r, t_r in ((wd0_ref, bd0_ref, gd0_ref, td0_ref),
                                   (wd1_ref, bd1_ref, gd1_ref, td1_ref),
                                   (wd2_ref, bd2_ref, gd2_ref, td2_ref)):
            u = jnp.dot(z, w_r[...], preferred_element_type=jnp.float32) + b_r[...]
            mu = jnp.mean(u, axis=0, keepdims=True)
            var = jnp.mean(u * u, axis=0, keepdims=True) - mu * mu
            z = jnp.maximum((u - mu) * lax.rsqrt(var + 1e-5) * g_r[...] + t_r[...], 0.0)
        z_ref[...] = z
        zmu_ref[...] = zmu
        zls_ref[...] = zls

    return pl.pallas_call(
        body,
        out_shape=(
            jax.ShapeDtypeStruct((B, H2d), jnp.float32),
            jax.ShapeDtypeStruct((B, ZD), jnp.float32),
            jax.ShapeDtypeStruct((B, ZD), jnp.float32),
        ),
    )


@functools.lru_cache(maxsize=None)
def _decoder_heads(B, HD, DX, DA, DE, Bb=256):
    grid = B // Bb

    def body(z_ref, wx_ref, bx_ref, wa_ref, ba_ref, we_ref, be_ref,
             xr_ref, a_ref, er_ref):
        z = z_ref[...]
        xr_ref[...] = jnp.dot(z, wx_ref[...], preferred_element_type=jnp.float32) + bx_ref[...]
        a_ref[...] = jnp.dot(z, wa_ref[...], preferred_element_type=jnp.float32) + ba_ref[...]
        er_ref[...] = jnp.dot(z, we_ref[...], preferred_element_type=jnp.float32) + be_ref[...]

    return pl.pallas_call(
        body,
        grid=(grid,),
        in_specs=[
            pl.BlockSpec((Bb, HD), lambda i: (i, 0)),
            pl.BlockSpec((HD, DX), lambda i: (0, 0)),
            pl.BlockSpec((1, DX), lambda i: (0, 0)),
            pl.BlockSpec((HD, DA), lambda i: (0, 0)),
            pl.BlockSpec((1, DA), lambda i: (0, 0)),
            pl.BlockSpec((HD, DE), lambda i: (0, 0)),
            pl.BlockSpec((1, DE), lambda i: (0, 0)),
        ],
        out_specs=(
            pl.BlockSpec((Bb, DX), lambda i: (i, 0)),
            pl.BlockSpec((Bb, DA), lambda i: (i, 0)),
            pl.BlockSpec((Bb, DE), lambda i: (i, 0)),
        ),
        out_shape=(
            jax.ShapeDtypeStruct((B, DX), jnp.float32),
            jax.ShapeDtypeStruct((B, DA), jnp.float32),
            jax.ShapeDtypeStruct((B, DE), jnp.float32),
        ),
    )


def kernel(x, edge_index, edge_attr, batch, edge_batch, eps, params):
    p = params
    N, IN = x.shape
    E = edge_attr.shape[0]
    B, ZD = eps.shape
    H1 = p["Wr1"].shape[1]
    H2 = p["Wr2"].shape[1]
    OUT = p["Ws"].shape[1]
    H2d = p["WX"].shape[0]
    DX = p["WX"].shape[1]
    DA = p["WA"].shape[1]
    DE = p["WE"].shape[1]

    src = edge_index[0].reshape(-1, _CHUNK)
    dst = edge_index[1].reshape(-1, _CHUNK)
    bat = batch.reshape(-1, _CHUNK)

    ea5 = jnp.concatenate([edge_attr.T, jnp.ones((1, E), jnp.float32)], axis=0)
    we1t = jnp.concatenate([p["We1"].T, p["be1"].reshape(-1, 1)], axis=1)
    we2t = jnp.concatenate([p["We2"].T, p["be2"].reshape(-1, 1)], axis=1)

    z1 = jnp.zeros((min(N // _NS, 256), H1), jnp.float32)
    z2 = jnp.zeros((min(N // _NS, 256), H2), jnp.float32)
    z3 = jnp.zeros((min(B // _NS, 256), OUT), jnp.float32)

    r2 = lambda a: a.reshape(1, -1)

    xg = _gather_rows(E, IN, N)(x, src)
    m1 = _edge_combine(E, IN, H1, Eb=4096)(xg, ea5, we1t)
    part1 = _scatter_add_rows(E, H1, N)(m1, dst, z1).reshape(2, N, H1)
    h1 = _bn_relu(N, IN, H1)(part1, x, p["Wr1"], r2(p["br1"]), r2(p["g1"]), r2(p["b1"]))

    hg = _gather_rows(E, H1, N)(h1, src)
    m2 = _edge_combine(E, H1, H2, Eb=4096)(hg, ea5, we2t)
    part2 = _scatter_add_rows(E, H2, N)(m2, dst, z2).reshape(2, N, H2)
    f_x = _bn_fx(N, H1, H2, OUT)(part2, h1, p["Wr2"], r2(p["br2"]), r2(p["g2"]),
                                 r2(p["b2"]), p["Ws"], r2(p["bs"]), p["Wt"], r2(p["bt"]))

    featp = _scatter_add_rows(N, OUT, B)(f_x, bat, z3).reshape(2, B, OUT)
    z, z_mu, z_lsgms = _decoder_trunk(B, OUT, ZD, *[p["Wd%d" % i].shape[1] for i in range(3)])(
        featp, eps,
        p["W11"], r2(p["b11"]), p["W12"], r2(p["b12"]),
        p["Wd0"], r2(p["bd0"]), r2(p["gd0"]), r2(p["betad0"]),
        p["Wd1"], r2(p["bd1"]), r2(p["gd1"]), r2(p["betad1"]),
        p["Wd2"], r2(p["bd2"]), r2(p["gd2"]), r2(p["betad2"]))
    xr, A, Er = _decoder_heads(B, H2d, DX, DA, DE)(
        z, p["WX"], r2(p["bX"]), p["WA"], r2(p["bA"]), p["WE"], r2(p["bE"]))

    return (xr.reshape(B, 38, 16), A, Er.reshape(B, DA, 4), z_mu, z_lsgms)
